# baseline (speedup 1.0000x reference)
import dataclasses
import functools

import jax
import jax.numpy as jnp
from jax import lax
from jax.experimental import pallas as pl
from jax.experimental.pallas import tpu as pltpu
from jax.experimental.pallas import tpu_sc as plsc

_T = 0.07
_B = 4096
_N = 100000
_D = 128
_RB = 512
_CB = 2048
_NCB = (_N + _CB - 1) // _CB

_NW = 32
_RPT = _B // _NW
_NV = _N // 16
_UNROLL = 25
_MININT = -2147483648

_mesh = plsc.VectorSubcoreMesh(core_axis_name="c", subcore_axis_name="s")
_CP = pltpu.CompilerParams()
if "needs_layout_passes" in pltpu.CompilerParams.__dataclass_fields__:
    _CP = dataclasses.replace(_CP, needs_layout_passes=False)


def _norm_body(x_ref, o_ref):
    x = x_ref[...]
    o_ref[...] = x / jnp.sqrt(jnp.sum(x * x, axis=1, keepdims=True))


def _normalize(points):
    return pl.pallas_call(
        _norm_body,
        out_shape=jax.ShapeDtypeStruct((_B, _D), jnp.float32),
    )(points)


def _matmul_body(np_ref, bank_ref, s_ref):
    s_ref[...] = lax.dot_general(
        np_ref[...], bank_ref[...], (((1,), (1,)), ((), ())),
        preferred_element_type=jnp.float32,
        precision=lax.Precision.HIGHEST)


def _similarities(norm_points, memory_bank):
    return pl.pallas_call(
        _matmul_body,
        grid=(_B // _RB, _NCB),
        in_specs=[
            pl.BlockSpec((_RB, _D), lambda i, j: (i, 0)),
            pl.BlockSpec((_CB, _D), lambda i, j: (j, 0)),
        ],
        out_specs=pl.BlockSpec((_RB, _CB), lambda i, j: (i, j)),
        out_shape=jax.ShapeDtypeStruct((_B, _N), jnp.float32),
    )(norm_points, memory_bank)


def _lane():
    return lax.iota(jnp.int32, 16)


def _key_of(v):
    b = plsc.bitcast(v, jnp.int32)
    return b ^ ((b >> 31) | jnp.int32(_MININT))


def _walk(hist, base, nsg, rank):
    big = jnp.int32(2147483647)

    def sg_body(s, sgv):
        a = jnp.zeros((16,), jnp.int32)
        for j in range(16):
            a = a + hist[pl.ds(base + s * 256 + j * 16, 16)]
        return jnp.where(_lane() == s, jnp.sum(a), sgv)

    sgv = lax.fori_loop(0, nsg, sg_body, jnp.zeros((16,), jnp.int32))
    tot = jnp.sum(sgv)
    suf = tot - plsc.cumsum(sgv) + sgv
    mask = suf >= rank
    s_star = jnp.max(jnp.where(mask, _lane(), 0))
    suf_s = jnp.min(jnp.where(mask, suf, big))
    above_sg = suf_s - jnp.sum(jnp.where(_lane() == s_star, sgv, 0))

    def vj_body(j, vjv):
        a = hist[pl.ds(base + s_star * 256 + j * 16, 16)]
        return jnp.where(_lane() == j, jnp.sum(a), vjv)

    vjv = lax.fori_loop(0, 16, vj_body, jnp.zeros((16,), jnp.int32))
    tot_v = jnp.sum(vjv)
    suf_v = tot_v - plsc.cumsum(vjv) + vjv
    rank_v = rank - above_sg
    mask_v = suf_v >= rank_v
    j_star = jnp.max(jnp.where(mask_v, _lane(), 0))
    suf_j = jnp.min(jnp.where(mask_v, suf_v, big))
    above_vj = suf_j - jnp.sum(jnp.where(_lane() == j_star, vjv, 0))

    lv = hist[pl.ds(base + s_star * 256 + j_star * 16, 16)]
    tot_l = jnp.sum(lv)
    suf_l = tot_l - plsc.cumsum(lv) + lv
    rank_l = rank_v - above_vj
    mask_l = suf_l >= rank_l
    l_star = jnp.max(jnp.where(mask_l, _lane(), 0))
    suf_ll = jnp.min(jnp.where(mask_l, suf_l, big))
    above_l = suf_ll - jnp.sum(jnp.where(_lane() == l_star, lv, 0))

    return (s_star * 256 + j_star * 16 + l_star,
            above_sg + above_vj + above_l)


def _clear(hist, base, nwords):
    z = jnp.zeros((16,), jnp.int32)
    for i in range(nwords // 16):
        hist[pl.ds(base + i * 16, 16)] = z


@functools.partial(
    pl.kernel,
    out_type=jax.ShapeDtypeStruct((_B * 16,), jnp.int32),
    mesh=_mesh,
    compiler_params=_CP,
    scratch_types=[
        pltpu.VMEM((_N,), jnp.float32),
        pltpu.VMEM((8192,), jnp.int32),
        pltpu.VMEM((_RPT * 16,), jnp.int32),
        pltpu.SemaphoreType.DMA,
    ],
)
def _select_sc(s_hbm, out_hbm, row_v, hist, res_v, sem):
    wid = lax.axis_index("s") * 2 + lax.axis_index("c")
    ones = jnp.ones((16,), jnp.int32)
    pltpu.async_copy(s_hbm.at[wid * _RPT], row_v, sem)

    @pl.loop(0, _RPT)
    def row_body(i):
        r = wid * _RPT + i
        pltpu.make_async_copy(s_hbm.at[r], row_v, sem).wait()

        _clear(hist, 0, 2048)

        @plsc.parallel_loop(0, _NV, unroll=_UNROLL)
        def scan0(q):
            key = _key_of(row_v[pl.ds(q * 16, 16)])
            plsc.addupdate_scatter(
                hist, [lax.shift_right_logical(key, 21)], ones)

        p4_0, a4_0 = _walk(hist, 0, 8, jnp.int32(4096))
        p1_0, a1_0 = _walk(hist, 0, 8, jnp.int32(100))
        r4 = 4096 - a4_0
        r1 = 100 - a1_0

        _clear(hist, 2048, 4096)

        @plsc.parallel_loop(0, _NV, unroll=_UNROLL)
        def scan1(q):
            key = _key_of(row_v[pl.ds(q * 16, 16)])
            top = lax.shift_right_logical(key, 21)
            mid = lax.shift_right_logical(key, 10) & 0x7FF
            plsc.addupdate_scatter(hist, [2048 + mid], ones,
                                   mask=top == p4_0)
            plsc.addupdate_scatter(hist, [4096 + mid], ones,
                                   mask=top == p1_0)

        p4_1, a4_1 = _walk(hist, 2048, 8, r4)
        p1_1, a1_1 = _walk(hist, 4096, 8, r1)
        r4b = r4 - a4_1
        r1b = r1 - a1_1
        pre4 = (p4_0 << 11) | p4_1
        pre1 = (p1_0 << 11) | p1_1

        _clear(hist, 6144, 2048)

        @plsc.parallel_loop(0, _NV, unroll=_UNROLL)
        def scan2(q):
            key = _key_of(row_v[pl.ds(q * 16, 16)])
            hi22 = lax.shift_right_logical(key, 10)
            lo = key & 0x3FF
            plsc.addupdate_scatter(hist, [6144 + lo], ones,
                                   mask=hi22 == pre4)
            plsc.addupdate_scatter(hist, [7168 + lo], ones,
                                   mask=hi22 == pre1)

        @pl.when(i < _RPT - 1)
        def _prefetch():
            pltpu.async_copy(s_hbm.at[r + 1], row_v, sem)

        p4_2, a4_2 = _walk(hist, 6144, 4, r4b)
        p1_2, a1_2 = _walk(hist, 7168, 4, r1b)

        key4 = (pre4 << 10) | p4_2
        key1 = (pre1 << 10) | p1_2
        m4 = r4b - a4_2
        m1 = r1b - a1_2

        l = _lane()
        res = jnp.where(l == 0, key4,
              jnp.where(l == 1, m4,
              jnp.where(l == 2, key1,
              jnp.where(l == 3, m1, 0))))
        res_v[pl.ds(i * 16, 16)] = res

    pltpu.sync_copy(res_v, out_hbm.at[pl.ds(wid * _RPT * 16, _RPT * 16)])


def _decode_key(k):
    bits = jnp.where(k < 0, k ^ jnp.int32(_MININT), ~k)
    return lax.bitcast_convert_type(bits, jnp.float32)


def _sums_body(s_ref, vk4_ref, vk1_ref, idx_ref, s4_ref, s1_ref, pos_ref):
    j = pl.program_id(1)

    @pl.when(j == 0)
    def _init():
        s4_ref[...] = jnp.zeros_like(s4_ref)
        s1_ref[...] = jnp.zeros_like(s1_ref)
        pos_ref[...] = jnp.zeros_like(pos_ref)

    v = s_ref[...]
    col = j * _CB + lax.broadcasted_iota(jnp.int32, (_RB, _CB), 1)
    valid = col < _N
    e = jnp.exp(v * (1.0 / _T))
    e4 = jnp.where(valid & (v > vk4_ref[...]), e, 0.0)
    e1 = jnp.where(valid & (v > vk1_ref[...]), e, 0.0)
    pv = jnp.where(col == idx_ref[...], v, 0.0)
    s4_ref[...] += jnp.sum(e4, axis=1, keepdims=True)
    s1_ref[...] += jnp.sum(e1, axis=1, keepdims=True)
    pos_ref[...] += jnp.sum(pv, axis=1, keepdims=True)


def _masked_sums(s, vk4096, vk100, point_indices):
    col1 = pl.BlockSpec((_RB, 1), lambda i, j: (i, 0))
    out_shape = jax.ShapeDtypeStruct((_B, 1), jnp.float32)
    return pl.pallas_call(
        _sums_body,
        grid=(_B // _RB, _NCB),
        in_specs=[
            pl.BlockSpec((_RB, _CB), lambda i, j: (i, j)),
            col1, col1, col1,
        ],
        out_specs=(col1, col1, col1),
        out_shape=(out_shape, out_shape, out_shape),
    )(s, vk4096, vk100, point_indices[:, None])


def _combine_body(pos_ref, s4_ref, s1_ref, k4_ref, m4_ref, k1_ref, m1_ref,
                  loss_ref):
    vk4 = _decode_key(k4_ref[...])
    vk1 = _decode_key(k1_ref[...])
    den = s4_ref[...] + m4_ref[...].astype(jnp.float32) * jnp.exp(vk4 * (1.0 / _T))
    num = (jnp.exp(pos_ref[...] * (1.0 / _T)) + s1_ref[...]
           + m1_ref[...].astype(jnp.float32) * jnp.exp(vk1 * (1.0 / _T)))
    per_row = -jnp.log(num / den + 1e-7)
    loss_ref[...] = jnp.sum(per_row, axis=0, keepdims=True) * (1.0 / _B)


def _combine(pos, s4, s1, k4, m4, k1, m1):
    out = pl.pallas_call(
        _combine_body,
        out_shape=jax.ShapeDtypeStruct((1, 1), jnp.float32),
    )(pos, s4, s1, k4, m4, k1, m1)
    return out.reshape(())


def kernel(points, point_indices, memory_bank):
    norm_points = _normalize(points)
    s = _similarities(norm_points, memory_bank)
    sel = _select_sc(s).reshape(_B, 16)
    k4, m4 = sel[:, 0:1], sel[:, 1:2]
    k1, m1 = sel[:, 2:3], sel[:, 3:4]
    vk4 = _decode_key(k4)
    vk1 = _decode_key(k1)
    s4, s1, pos = _masked_sums(s, vk4, vk1, point_indices.astype(jnp.int32))
    loss = _combine(pos, s4, s1, k4, m4, k1, m1)
    return (loss, s)


# speedup vs baseline: 1.6839x; 1.6839x over previous
import dataclasses
import functools

import jax
import jax.numpy as jnp
from jax import lax
from jax.experimental import pallas as pl
from jax.experimental.pallas import tpu as pltpu
from jax.experimental.pallas import tpu_sc as plsc

_T = 0.07
_B = 4096
_N = 100000
_D = 128
_RB = 512
_CB = 2048
_NCB = (_N + _CB - 1) // _CB

_NW = 32
_RPT = _B // _NW
_NV = _N // 16
_UNROLL = 5
_MININT = -2147483648

_mesh = plsc.VectorSubcoreMesh(core_axis_name="c", subcore_axis_name="s")
_CP = pltpu.CompilerParams()
if "needs_layout_passes" in pltpu.CompilerParams.__dataclass_fields__:
    _CP = dataclasses.replace(_CP, needs_layout_passes=False)


def _norm_body(x_ref, o_ref):
    x = x_ref[...]
    o_ref[...] = x / jnp.sqrt(jnp.sum(x * x, axis=1, keepdims=True))


def _normalize(points):
    return pl.pallas_call(
        _norm_body,
        out_shape=jax.ShapeDtypeStruct((_B, _D), jnp.float32),
    )(points)


def _matmul_body(np_ref, bank_ref, s_ref):
    s_ref[...] = lax.dot_general(
        np_ref[...], bank_ref[...], (((1,), (1,)), ((), ())),
        preferred_element_type=jnp.float32,
        precision=lax.Precision.HIGHEST)


def _similarities(norm_points, memory_bank):
    return pl.pallas_call(
        _matmul_body,
        grid=(_B // _RB, _NCB),
        in_specs=[
            pl.BlockSpec((_RB, _D), lambda i, j: (i, 0)),
            pl.BlockSpec((_CB, _D), lambda i, j: (j, 0)),
        ],
        out_specs=pl.BlockSpec((_RB, _CB), lambda i, j: (i, j)),
        out_shape=jax.ShapeDtypeStruct((_B, _N), jnp.float32),
    )(norm_points, memory_bank)


def _lane():
    return lax.iota(jnp.int32, 16)


def _key_of(v):
    b = plsc.bitcast(v, jnp.int32)
    return b ^ ((b >> 31) | jnp.int32(_MININT))


def _walk(hist, base, nsg, rank):
    big = jnp.int32(2147483647)

    def sg_body(s, sgv):
        a = jnp.zeros((16,), jnp.int32)
        for j in range(16):
            a = a + hist[pl.ds(base + s * 256 + j * 16, 16)]
        return jnp.where(_lane() == s, jnp.sum(a), sgv)

    sgv = lax.fori_loop(0, nsg, sg_body, jnp.zeros((16,), jnp.int32))
    tot = jnp.sum(sgv)
    suf = tot - plsc.cumsum(sgv) + sgv
    mask = suf >= rank
    s_star = jnp.max(jnp.where(mask, _lane(), 0))
    suf_s = jnp.min(jnp.where(mask, suf, big))
    above_sg = suf_s - jnp.sum(jnp.where(_lane() == s_star, sgv, 0))

    def vj_body(j, vjv):
        a = hist[pl.ds(base + s_star * 256 + j * 16, 16)]
        return jnp.where(_lane() == j, jnp.sum(a), vjv)

    vjv = lax.fori_loop(0, 16, vj_body, jnp.zeros((16,), jnp.int32))
    tot_v = jnp.sum(vjv)
    suf_v = tot_v - plsc.cumsum(vjv) + vjv
    rank_v = rank - above_sg
    mask_v = suf_v >= rank_v
    j_star = jnp.max(jnp.where(mask_v, _lane(), 0))
    suf_j = jnp.min(jnp.where(mask_v, suf_v, big))
    above_vj = suf_j - jnp.sum(jnp.where(_lane() == j_star, vjv, 0))

    lv = hist[pl.ds(base + s_star * 256 + j_star * 16, 16)]
    tot_l = jnp.sum(lv)
    suf_l = tot_l - plsc.cumsum(lv) + lv
    rank_l = rank_v - above_vj
    mask_l = suf_l >= rank_l
    l_star = jnp.max(jnp.where(mask_l, _lane(), 0))
    suf_ll = jnp.min(jnp.where(mask_l, suf_l, big))
    above_l = suf_ll - jnp.sum(jnp.where(_lane() == l_star, lv, 0))

    return (s_star * 256 + j_star * 16 + l_star,
            above_sg + above_vj + above_l)


def _clear(hist, base, nwords):
    z = jnp.zeros((16,), jnp.int32)
    for i in range(nwords // 16):
        hist[pl.ds(base + i * 16, 16)] = z


@functools.partial(
    pl.kernel,
    out_type=jax.ShapeDtypeStruct((_B * 16,), jnp.int32),
    mesh=_mesh,
    compiler_params=_CP,
    scratch_types=[
        pltpu.VMEM((_N,), jnp.float32),
        pltpu.VMEM((8192,), jnp.int32),
        pltpu.VMEM((_RPT * 16,), jnp.int32),
        pltpu.SemaphoreType.DMA,
    ],
)
def _select_sc(s_hbm, out_hbm, row_v, hist, res_v, sem):
    wid = lax.axis_index("s") * 2 + lax.axis_index("c")
    ones = jnp.ones((16,), jnp.int32)
    pltpu.async_copy(s_hbm.at[wid * _RPT], row_v, sem)

    @pl.loop(0, _RPT)
    def row_body(i):
        r = wid * _RPT + i
        pltpu.make_async_copy(s_hbm.at[r], row_v, sem).wait()

        _clear(hist, 0, 2048)

        @plsc.parallel_loop(0, _NV, unroll=_UNROLL)
        def scan0(q):
            key = _key_of(row_v[pl.ds(q * 16, 16)])
            plsc.addupdate_scatter(
                hist, [lax.shift_right_logical(key, 21)], ones)

        p4_0, a4_0 = _walk(hist, 0, 8, jnp.int32(4096))
        p1_0, a1_0 = _walk(hist, 0, 8, jnp.int32(100))
        r4 = 4096 - a4_0
        r1 = 100 - a1_0

        _clear(hist, 2048, 4096)

        @plsc.parallel_loop(0, _NV, unroll=_UNROLL)
        def scan1(q):
            key = _key_of(row_v[pl.ds(q * 16, 16)])
            top = lax.shift_right_logical(key, 21)
            mid = lax.shift_right_logical(key, 10) & 0x7FF
            plsc.addupdate_scatter(hist, [2048 + mid], ones,
                                   mask=top == p4_0)
            plsc.addupdate_scatter(hist, [4096 + mid], ones,
                                   mask=top == p1_0)

        p4_1, a4_1 = _walk(hist, 2048, 8, r4)
        p1_1, a1_1 = _walk(hist, 4096, 8, r1)
        r4b = r4 - a4_1
        r1b = r1 - a1_1
        pre4 = (p4_0 << 11) | p4_1
        pre1 = (p1_0 << 11) | p1_1

        _clear(hist, 6144, 2048)

        @plsc.parallel_loop(0, _NV, unroll=_UNROLL)
        def scan2(q):
            key = _key_of(row_v[pl.ds(q * 16, 16)])
            hi22 = lax.shift_right_logical(key, 10)
            lo = key & 0x3FF
            plsc.addupdate_scatter(hist, [6144 + lo], ones,
                                   mask=hi22 == pre4)
            plsc.addupdate_scatter(hist, [7168 + lo], ones,
                                   mask=hi22 == pre1)

        @pl.when(i < _RPT - 1)
        def _prefetch():
            pltpu.async_copy(s_hbm.at[r + 1], row_v, sem)

        p4_2, a4_2 = _walk(hist, 6144, 4, r4b)
        p1_2, a1_2 = _walk(hist, 7168, 4, r1b)

        key4 = (pre4 << 10) | p4_2
        key1 = (pre1 << 10) | p1_2
        m4 = r4b - a4_2
        m1 = r1b - a1_2

        l = _lane()
        res = jnp.where(l == 0, key4,
              jnp.where(l == 1, m4,
              jnp.where(l == 2, key1,
              jnp.where(l == 3, m1, 0))))
        res_v[pl.ds(i * 16, 16)] = res

    pltpu.sync_copy(res_v, out_hbm.at[pl.ds(wid * _RPT * 16, _RPT * 16)])


def _decode_key(k):
    bits = jnp.where(k < 0, k ^ jnp.int32(_MININT), ~k)
    return lax.bitcast_convert_type(bits, jnp.float32)


def _sums_body(s_ref, vk4_ref, vk1_ref, idx_ref, s4_ref, s1_ref, pos_ref):
    j = pl.program_id(1)

    @pl.when(j == 0)
    def _init():
        s4_ref[...] = jnp.zeros_like(s4_ref)
        s1_ref[...] = jnp.zeros_like(s1_ref)
        pos_ref[...] = jnp.zeros_like(pos_ref)

    v = s_ref[...]
    col = j * _CB + lax.broadcasted_iota(jnp.int32, (_RB, _CB), 1)
    valid = col < _N
    e = jnp.exp(v * (1.0 / _T))
    e4 = jnp.where(valid & (v > vk4_ref[...]), e, 0.0)
    e1 = jnp.where(valid & (v > vk1_ref[...]), e, 0.0)
    pv = jnp.where(col == idx_ref[...], v, 0.0)
    s4_ref[...] += jnp.sum(e4, axis=1, keepdims=True)
    s1_ref[...] += jnp.sum(e1, axis=1, keepdims=True)
    pos_ref[...] += jnp.sum(pv, axis=1, keepdims=True)


def _masked_sums(s, vk4096, vk100, point_indices):
    col1 = pl.BlockSpec((_RB, 1), lambda i, j: (i, 0))
    out_shape = jax.ShapeDtypeStruct((_B, 1), jnp.float32)
    return pl.pallas_call(
        _sums_body,
        grid=(_B // _RB, _NCB),
        in_specs=[
            pl.BlockSpec((_RB, _CB), lambda i, j: (i, j)),
            col1, col1, col1,
        ],
        out_specs=(col1, col1, col1),
        out_shape=(out_shape, out_shape, out_shape),
    )(s, vk4096, vk100, point_indices[:, None])


def _combine_body(pos_ref, s4_ref, s1_ref, k4_ref, m4_ref, k1_ref, m1_ref,
                  loss_ref):
    vk4 = _decode_key(k4_ref[...])
    vk1 = _decode_key(k1_ref[...])
    den = s4_ref[...] + m4_ref[...].astype(jnp.float32) * jnp.exp(vk4 * (1.0 / _T))
    num = (jnp.exp(pos_ref[...] * (1.0 / _T)) + s1_ref[...]
           + m1_ref[...].astype(jnp.float32) * jnp.exp(vk1 * (1.0 / _T)))
    per_row = -jnp.log(num / den + 1e-7)
    loss_ref[...] = jnp.sum(per_row, axis=0, keepdims=True) * (1.0 / _B)


def _combine(pos, s4, s1, k4, m4, k1, m1):
    out = pl.pallas_call(
        _combine_body,
        out_shape=jax.ShapeDtypeStruct((1, 1), jnp.float32),
    )(pos, s4, s1, k4, m4, k1, m1)
    return out.reshape(())


def kernel(points, point_indices, memory_bank):
    norm_points = _normalize(points)
    s = _similarities(norm_points, memory_bank)
    sel = _select_sc(s).reshape(_B, 16)
    k4, m4 = sel[:, 0:1], sel[:, 1:2]
    k1, m1 = sel[:, 2:3], sel[:, 3:4]
    vk4 = _decode_key(k4)
    vk1 = _decode_key(k1)
    s4, s1, pos = _masked_sums(s, vk4, vk1, point_indices.astype(jnp.int32))
    loss = _combine(pos, s4, s1, k4, m4, k1, m1)
    return (loss, s)


# speedup vs baseline: 1.8585x; 1.1037x over previous
import dataclasses
import functools

import jax
import jax.numpy as jnp
from jax import lax
from jax.experimental import pallas as pl
from jax.experimental.pallas import tpu as pltpu
from jax.experimental.pallas import tpu_sc as plsc

_T = 0.07
_B = 4096
_N = 100000
_D = 128
_RB = 512
_CB = 2048
_NCB = (_N + _CB - 1) // _CB

_NW = 32
_RPT = _B // _NW
_NV = _N // 16
_UNROLL = 5
_MININT = -2147483648

_mesh = plsc.VectorSubcoreMesh(core_axis_name="c", subcore_axis_name="s")
_CP = pltpu.CompilerParams()
if "needs_layout_passes" in pltpu.CompilerParams.__dataclass_fields__:
    _CP = dataclasses.replace(_CP, needs_layout_passes=False)


def _norm_body(x_ref, o_ref):
    x = x_ref[...]
    o_ref[...] = x / jnp.sqrt(jnp.sum(x * x, axis=1, keepdims=True))


def _normalize(points):
    return pl.pallas_call(
        _norm_body,
        out_shape=jax.ShapeDtypeStruct((_B, _D), jnp.float32),
    )(points)


def _matmul_body(np_ref, bank_ref, s_ref):
    s_ref[...] = lax.dot_general(
        np_ref[...], bank_ref[...], (((1,), (1,)), ((), ())),
        preferred_element_type=jnp.float32,
        precision=lax.Precision.DEFAULT)


def _similarities(norm_points, memory_bank):
    return pl.pallas_call(
        _matmul_body,
        grid=(_B // _RB, _NCB),
        in_specs=[
            pl.BlockSpec((_RB, _D), lambda i, j: (i, 0)),
            pl.BlockSpec((_CB, _D), lambda i, j: (j, 0)),
        ],
        out_specs=pl.BlockSpec((_RB, _CB), lambda i, j: (i, j)),
        out_shape=jax.ShapeDtypeStruct((_B, _N), jnp.float32),
    )(norm_points, memory_bank)


def _lane():
    return lax.iota(jnp.int32, 16)


def _key_of(v):
    b = plsc.bitcast(v, jnp.int32)
    return b ^ ((b >> 31) | jnp.int32(_MININT))


def _walk(hist, base, nsg, rank):
    big = jnp.int32(2147483647)

    def sg_body(s, sgv):
        a = jnp.zeros((16,), jnp.int32)
        for j in range(16):
            a = a + hist[pl.ds(base + s * 256 + j * 16, 16)]
        return jnp.where(_lane() == s, jnp.sum(a), sgv)

    sgv = lax.fori_loop(0, nsg, sg_body, jnp.zeros((16,), jnp.int32))
    tot = jnp.sum(sgv)
    suf = tot - plsc.cumsum(sgv) + sgv
    mask = suf >= rank
    s_star = jnp.max(jnp.where(mask, _lane(), 0))
    suf_s = jnp.min(jnp.where(mask, suf, big))
    above_sg = suf_s - jnp.sum(jnp.where(_lane() == s_star, sgv, 0))

    def vj_body(j, vjv):
        a = hist[pl.ds(base + s_star * 256 + j * 16, 16)]
        return jnp.where(_lane() == j, jnp.sum(a), vjv)

    vjv = lax.fori_loop(0, 16, vj_body, jnp.zeros((16,), jnp.int32))
    tot_v = jnp.sum(vjv)
    suf_v = tot_v - plsc.cumsum(vjv) + vjv
    rank_v = rank - above_sg
    mask_v = suf_v >= rank_v
    j_star = jnp.max(jnp.where(mask_v, _lane(), 0))
    suf_j = jnp.min(jnp.where(mask_v, suf_v, big))
    above_vj = suf_j - jnp.sum(jnp.where(_lane() == j_star, vjv, 0))

    lv = hist[pl.ds(base + s_star * 256 + j_star * 16, 16)]
    tot_l = jnp.sum(lv)
    suf_l = tot_l - plsc.cumsum(lv) + lv
    rank_l = rank_v - above_vj
    mask_l = suf_l >= rank_l
    l_star = jnp.max(jnp.where(mask_l, _lane(), 0))
    suf_ll = jnp.min(jnp.where(mask_l, suf_l, big))
    above_l = suf_ll - jnp.sum(jnp.where(_lane() == l_star, lv, 0))

    return (s_star * 256 + j_star * 16 + l_star,
            above_sg + above_vj + above_l)


def _clear(hist, base, nwords):
    z = jnp.zeros((16,), jnp.int32)
    for i in range(nwords // 16):
        hist[pl.ds(base + i * 16, 16)] = z


@functools.partial(
    pl.kernel,
    out_type=jax.ShapeDtypeStruct((_B * 16,), jnp.int32),
    mesh=_mesh,
    compiler_params=_CP,
    scratch_types=[
        pltpu.VMEM((_N,), jnp.float32),
        pltpu.VMEM((8192,), jnp.int32),
        pltpu.VMEM((_RPT * 16,), jnp.int32),
        pltpu.SemaphoreType.DMA,
    ],
)
def _select_sc(s_hbm, out_hbm, row_v, hist, res_v, sem):
    wid = lax.axis_index("s") * 2 + lax.axis_index("c")
    ones = jnp.ones((16,), jnp.int32)
    pltpu.async_copy(s_hbm.at[wid * _RPT], row_v, sem)

    @pl.loop(0, _RPT)
    def row_body(i):
        r = wid * _RPT + i
        pltpu.make_async_copy(s_hbm.at[r], row_v, sem).wait()

        _clear(hist, 0, 2048)

        @plsc.parallel_loop(0, _NV, unroll=_UNROLL)
        def scan0(q):
            key = _key_of(row_v[pl.ds(q * 16, 16)])
            plsc.addupdate_scatter(
                hist, [lax.shift_right_logical(key, 21)], ones)

        p4_0, a4_0 = _walk(hist, 0, 8, jnp.int32(4096))
        p1_0, a1_0 = _walk(hist, 0, 8, jnp.int32(100))
        r4 = 4096 - a4_0
        r1 = 100 - a1_0

        _clear(hist, 2048, 4096)

        @plsc.parallel_loop(0, _NV, unroll=_UNROLL)
        def scan1(q):
            key = _key_of(row_v[pl.ds(q * 16, 16)])
            top = lax.shift_right_logical(key, 21)
            mid = lax.shift_right_logical(key, 10) & 0x7FF
            plsc.addupdate_scatter(hist, [2048 + mid], ones,
                                   mask=top == p4_0)
            plsc.addupdate_scatter(hist, [4096 + mid], ones,
                                   mask=top == p1_0)

        p4_1, a4_1 = _walk(hist, 2048, 8, r4)
        p1_1, a1_1 = _walk(hist, 4096, 8, r1)
        r4b = r4 - a4_1
        r1b = r1 - a1_1
        pre4 = (p4_0 << 11) | p4_1
        pre1 = (p1_0 << 11) | p1_1

        _clear(hist, 6144, 2048)

        @plsc.parallel_loop(0, _NV, unroll=_UNROLL)
        def scan2(q):
            key = _key_of(row_v[pl.ds(q * 16, 16)])
            hi22 = lax.shift_right_logical(key, 10)
            lo = key & 0x3FF
            plsc.addupdate_scatter(hist, [6144 + lo], ones,
                                   mask=hi22 == pre4)
            plsc.addupdate_scatter(hist, [7168 + lo], ones,
                                   mask=hi22 == pre1)

        @pl.when(i < _RPT - 1)
        def _prefetch():
            pltpu.async_copy(s_hbm.at[r + 1], row_v, sem)

        p4_2, a4_2 = _walk(hist, 6144, 4, r4b)
        p1_2, a1_2 = _walk(hist, 7168, 4, r1b)

        key4 = (pre4 << 10) | p4_2
        key1 = (pre1 << 10) | p1_2
        m4 = r4b - a4_2
        m1 = r1b - a1_2

        l = _lane()
        res = jnp.where(l == 0, key4,
              jnp.where(l == 1, m4,
              jnp.where(l == 2, key1,
              jnp.where(l == 3, m1, 0))))
        res_v[pl.ds(i * 16, 16)] = res

    pltpu.sync_copy(res_v, out_hbm.at[pl.ds(wid * _RPT * 16, _RPT * 16)])


def _decode_key(k):
    bits = jnp.where(k < 0, k ^ jnp.int32(_MININT), ~k)
    return lax.bitcast_convert_type(bits, jnp.float32)


def _sums_body(s_ref, vk4_ref, vk1_ref, idx_ref, s4_ref, s1_ref, pos_ref):
    j = pl.program_id(1)

    @pl.when(j == 0)
    def _init():
        s4_ref[...] = jnp.zeros_like(s4_ref)
        s1_ref[...] = jnp.zeros_like(s1_ref)
        pos_ref[...] = jnp.zeros_like(pos_ref)

    v = s_ref[...]
    col = j * _CB + lax.broadcasted_iota(jnp.int32, (_RB, _CB), 1)
    valid = col < _N
    e = jnp.exp(v * (1.0 / _T))
    e4 = jnp.where(valid & (v > vk4_ref[...]), e, 0.0)
    e1 = jnp.where(valid & (v > vk1_ref[...]), e, 0.0)
    pv = jnp.where(col == idx_ref[...], v, 0.0)
    s4_ref[...] += jnp.sum(e4, axis=1, keepdims=True)
    s1_ref[...] += jnp.sum(e1, axis=1, keepdims=True)
    pos_ref[...] += jnp.sum(pv, axis=1, keepdims=True)


def _masked_sums(s, vk4096, vk100, point_indices):
    col1 = pl.BlockSpec((_RB, 1), lambda i, j: (i, 0))
    out_shape = jax.ShapeDtypeStruct((_B, 1), jnp.float32)
    return pl.pallas_call(
        _sums_body,
        grid=(_B // _RB, _NCB),
        in_specs=[
            pl.BlockSpec((_RB, _CB), lambda i, j: (i, j)),
            col1, col1, col1,
        ],
        out_specs=(col1, col1, col1),
        out_shape=(out_shape, out_shape, out_shape),
    )(s, vk4096, vk100, point_indices[:, None])


def _combine_body(pos_ref, s4_ref, s1_ref, k4_ref, m4_ref, k1_ref, m1_ref,
                  loss_ref):
    vk4 = _decode_key(k4_ref[...])
    vk1 = _decode_key(k1_ref[...])
    den = s4_ref[...] + m4_ref[...].astype(jnp.float32) * jnp.exp(vk4 * (1.0 / _T))
    num = (jnp.exp(pos_ref[...] * (1.0 / _T)) + s1_ref[...]
           + m1_ref[...].astype(jnp.float32) * jnp.exp(vk1 * (1.0 / _T)))
    per_row = -jnp.log(num / den + 1e-7)
    loss_ref[...] = jnp.sum(per_row, axis=0, keepdims=True) * (1.0 / _B)


def _combine(pos, s4, s1, k4, m4, k1, m1):
    out = pl.pallas_call(
        _combine_body,
        out_shape=jax.ShapeDtypeStruct((1, 1), jnp.float32),
    )(pos, s4, s1, k4, m4, k1, m1)
    return out.reshape(())


def kernel(points, point_indices, memory_bank):
    norm_points = _normalize(points)
    s = _similarities(norm_points, memory_bank)
    sel = _select_sc(s).reshape(_B, 16)
    k4, m4 = sel[:, 0:1], sel[:, 1:2]
    k1, m1 = sel[:, 2:3], sel[:, 3:4]
    vk4 = _decode_key(k4)
    vk1 = _decode_key(k1)
    s4, s1, pos = _masked_sums(s, vk4, vk1, point_indices.astype(jnp.int32))
    loss = _combine(pos, s4, s1, k4, m4, k1, m1)
    return (loss, s)
